# baseline (speedup 1.0000x reference)
import functools

import jax
import jax.numpy as jnp
from jax import lax
from jax.experimental import pallas as pl
from jax.experimental.pallas import tpu as pltpu
from jax.experimental.pallas import tpu_sc as plsc

L = 16
NS = 16
CHUNK = 128


@functools.lru_cache(maxsize=None)
def _build(B):
    bpt = B // NS
    mesh = plsc.VectorSubcoreMesh(core_axis_name="c", subcore_axis_name="s")

    @functools.partial(
        pl.kernel,
        mesh=mesh,
        out_type=[
            jax.ShapeDtypeStruct((B,), jnp.float32),
            jax.ShapeDtypeStruct((B,), jnp.float32),
            jax.ShapeDtypeStruct((B,), jnp.float32),
            jax.ShapeDtypeStruct((B,), jnp.float32),
        ],
        scratch_types=[
            pltpu.VMEM((bpt,), jnp.int32),
            pltpu.VMEM((bpt,), jnp.uint32),
            pltpu.VMEM((2, bpt), jnp.uint32),
            pltpu.VMEM((2, bpt), jnp.float32),
            pltpu.VMEM((L,), jnp.float32),
            pltpu.VMEM((NS * L,), jnp.float32),
            pltpu.VMEM_SHARED((NS * L,), jnp.float32),
            pltpu.SemaphoreType.DMA,
        ],
    )
    def sc_kernel(uidx_hbm, wpp_hbm, wtp_hbm,
                  out_pp0, out_pp1, out_tp0, out_tp1,
                  uidx_v, words, ints, rows, accbuf, allp, shared, sem):
        c = lax.axis_index("c")
        sid = lax.axis_index("s")
        base = sid * bpt

        pltpu.sync_copy(uidx_hbm.at[pl.ds(base, bpt)], uidx_v)

        def gather(wtab):
            cps = [
                pltpu.async_copy(wtab.at[uidx_v.at[pl.ds(j * CHUNK, CHUNK)]],
                                 words.at[pl.ds(j * CHUNK, CHUNK)], sem)
                for j in range(bpt // CHUNK)
            ]
            for cp in cps:
                cp.wait()

        pl.when(c == 0)(lambda: gather(wpp_hbm))
        pl.when(c == 1)(lambda: gather(wtp_hbm))

        himask = jnp.uint32(0xFFFF0000)
        for q in range(bpt // L):
            w = words[pl.ds(q * L, L)]
            ints[0, pl.ds(q * L, L)] = w & himask
            ints[1, pl.ds(q * L, L)] = w << 16
        intsf = ints.bitcast(jnp.float32)

        acc = jnp.zeros((L,), jnp.float32)
        for h in range(2):
            for q in range(bpt // L):
                acc = acc + intsf[h, pl.ds(q * L, L)]
        accbuf[...] = acc

        pltpu.sync_copy(accbuf, shared.at[pl.ds(sid * L, L)])
        plsc.subcore_barrier()
        pltpu.sync_copy(shared, allp)

        tot = jnp.zeros((L,), jnp.float32)
        for t in range(NS):
            tot = tot + allp[pl.ds(t * L, L)]
        total = tot[0]
        for l in range(1, L):
            total = total + tot[l]

        for h in range(2):
            for q in range(bpt // L):
                rows[h, pl.ds(q * L, L)] = intsf[h, pl.ds(q * L, L)] / total

        def emit(o0, o1):
            pltpu.sync_copy(rows.at[0], o0.at[pl.ds(base, bpt)])
            pltpu.sync_copy(rows.at[1], o1.at[pl.ds(base, bpt)])

        pl.when(c == 0)(lambda: emit(out_pp0, out_pp1))
        pl.when(c == 1)(lambda: emit(out_tp0, out_tp1))

    return sc_kernel


def kernel(user_idx, item_idx, transition_preference, prob_preference,
           m1_user, m1_item, m2_user, m2_item):
    B = user_idx.shape[0]

    def pack(tab):
        bits = jax.lax.bitcast_convert_type(tab, jnp.uint32)
        rb = (bits + jnp.uint32(0x7FFF) + ((bits >> 16) & 1)) >> 16
        return jnp.sum(rb * jnp.array([[65536, 1]], jnp.uint32), axis=1)

    pp0, pp1, tp0, tp1 = _build(B)(
        user_idx.astype(jnp.int32),
        pack(prob_preference), pack(transition_preference),
    )
    return (jnp.stack([pp0, pp1], axis=-1), jnp.stack([tp0, tp1], axis=-1))


# speedup vs baseline: 1.0250x; 1.0250x over previous
import functools

import jax
import jax.numpy as jnp
from jax import lax
from jax.experimental import pallas as pl
from jax.experimental.pallas import tpu as pltpu
from jax.experimental.pallas import tpu_sc as plsc

L = 16
NS = 16
CHUNK = 128


@functools.lru_cache(maxsize=None)
def _build(B):
    bpt = B // NS
    E = 2 * bpt
    NROW = E // CHUNK
    mesh = plsc.VectorSubcoreMesh(core_axis_name="c", subcore_axis_name="s")

    @functools.partial(
        pl.kernel,
        mesh=mesh,
        out_type=[
            jax.ShapeDtypeStruct((B,), jnp.float32),
            jax.ShapeDtypeStruct((B,), jnp.float32),
            jax.ShapeDtypeStruct((B,), jnp.float32),
            jax.ShapeDtypeStruct((B,), jnp.float32),
        ],
        scratch_types=[
            pltpu.VMEM((bpt,), jnp.int32),
            pltpu.VMEM((E,), jnp.float32),
            pltpu.VMEM((L,), jnp.float32),
            pltpu.VMEM((NS * L,), jnp.float32),
            pltpu.VMEM_SHARED((NS * L,), jnp.float32),
            pltpu.SemaphoreType.DMA,
        ],
    )
    def sc_kernel(uidx_hbm, pp0_hbm, pp1_hbm, tp0_hbm, tp1_hbm,
                  out_pp0, out_pp1, out_tp0, out_tp1,
                  uidx_v, rows, accbuf, allp, shared, sem):
        c = lax.axis_index("c")
        sid = lax.axis_index("s")
        base = sid * bpt

        pltpu.sync_copy(uidx_hbm.at[pl.ds(base, bpt)], uidx_v)

        def gather(col0, col1):
            cps = [
                pltpu.async_copy(col.at[uidx_v.at[pl.ds(j * CHUNK, CHUNK)]],
                                 rows.at[pl.ds(h * bpt + j * CHUNK, CHUNK)],
                                 sem)
                for h, col in ((0, col0), (1, col1))
                for j in range(bpt // CHUNK)
            ]
            for cp in cps:
                cp.wait()

        pl.when(c == 0)(lambda: gather(pp0_hbm, pp1_hbm))
        pl.when(c == 1)(lambda: gather(tp0_hbm, tp1_hbm))

        acc = jnp.zeros((L,), jnp.float32)
        for q in range(E // L):
            acc = acc + rows[pl.ds(q * L, L)]
        accbuf[...] = acc

        pltpu.sync_copy(accbuf, shared.at[pl.ds(sid * L, L)])
        plsc.subcore_barrier()
        pltpu.sync_copy(shared, allp)

        tot = jnp.zeros((L,), jnp.float32)
        for t in range(NS):
            tot = tot + allp[pl.ds(t * L, L)]
        total = tot[0]
        for l in range(1, L):
            total = total + tot[l]

        for q in range(E // L):
            rows[pl.ds(q * L, L)] = rows[pl.ds(q * L, L)] / total

        def emit(o0, o1):
            pltpu.sync_copy(rows.at[pl.ds(0, bpt)], o0.at[pl.ds(base, bpt)])
            pltpu.sync_copy(rows.at[pl.ds(bpt, bpt)], o1.at[pl.ds(base, bpt)])

        pl.when(c == 0)(lambda: emit(out_pp0, out_pp1))
        pl.when(c == 1)(lambda: emit(out_tp0, out_tp1))

    return sc_kernel


def kernel(user_idx, item_idx, transition_preference, prob_preference,
           m1_user, m1_item, m2_user, m2_item):
    B = user_idx.shape[0]
    pp0, pp1, tp0, tp1 = _build(B)(
        user_idx.astype(jnp.int32),
        prob_preference[:, 0], prob_preference[:, 1],
        transition_preference[:, 0], transition_preference[:, 1],
    )
    return (jnp.stack([pp0, pp1], axis=-1), jnp.stack([tp0, tp1], axis=-1))
